# baseline (speedup 1.0000x reference)
import jax
import jax.numpy as jnp
from jax import lax
from jax.experimental import pallas as pl
from jax.experimental.pallas import tpu as pltpu
from jax.experimental.pallas import tpu_sc as plsc

N = 10000
E = 160000
H = 8
C = 16
D = 128
EH = 64

NC = 2
NS = 16
NW = NC * NS
B = 128
E_PAD = 163840
EW = E_PAD // NW
NB = EW // B
NP = 10240
ROWS_PER_SUB = NP // NS


def _ln_relu(y, g, b):
    mu = jnp.mean(y, axis=-1, keepdims=True)
    var = jnp.mean((y - mu) ** 2, axis=-1, keepdims=True)
    return jnp.maximum((y - mu) / jnp.sqrt(var + 1e-5) * g + b, 0.0)


def _node_enc_body(x_ref, w_ref, b_ref, g_ref, beta_ref, o_ref):
    y = x_ref[...] @ w_ref[...] + b_ref[...]
    o_ref[...] = _ln_relu(y, g_ref[...], beta_ref[...])


def _node_encoder(x_pad, ne_W, ne_b, ne_g, ne_beta):
    return pl.pallas_call(
        _node_enc_body,
        out_shape=jax.ShapeDtypeStruct((NP, D), jnp.float32),
    )(x_pad, ne_W, ne_b, ne_g, ne_beta)


def _edge_encoder(edge_attr_pad, ee_W, ee_b, ee_g, ee_beta):
    blk = 8192
    return pl.pallas_call(
        _node_enc_body,
        out_shape=jax.ShapeDtypeStruct((E_PAD, EH), jnp.float32),
        grid=(E_PAD // blk,),
        in_specs=[
            pl.BlockSpec((blk, 3), lambda i: (i, 0)),
            pl.BlockSpec((3, EH), lambda i: (0, 0)),
            pl.BlockSpec((EH,), lambda i: (0,)),
            pl.BlockSpec((EH,), lambda i: (0,)),
            pl.BlockSpec((EH,), lambda i: (0,)),
        ],
        out_specs=pl.BlockSpec((blk, EH), lambda i: (i, 0)),
    )(edge_attr_pad, ee_W, ee_b, ee_g, ee_beta)


def _proj_body(h_ref, wl_ref, bl_ref, wr_ref, br_ref, xl_ref, xr_ref):
    h = h_ref[...]
    xl_ref[...] = h @ wl_ref[...] + bl_ref[...]
    xr_ref[...] = h @ wr_ref[...] + br_ref[...]


def _projections(h, Wl_l, bl_l, Wr_l, br_l):
    return pl.pallas_call(
        _proj_body,
        out_shape=(
            jax.ShapeDtypeStruct((NP, D), jnp.float32),
            jax.ShapeDtypeStruct((NP, D), jnp.float32),
        ),
    )(h, Wl_l, bl_l, Wr_l, br_l)


def _edge_math_body(xls_ref, xrd_ref, ef_ref, we_ref, att_ref,
                    om_ref, od_ref):
    xls = xls_ref[0]
    ee = ef_ref[...] @ we_ref[...]
    m = xls + xrd_ref[0] + ee
    m = jnp.maximum(m, 0.2 * m)
    t = m * att_ref[...]
    ii = lax.broadcasted_iota(jnp.int32, (D, H), 1)
    jj = lax.broadcasted_iota(jnp.int32, (D, H), 0)
    gsel = (jj // C == ii).astype(jnp.float32)
    ex = jnp.exp(t @ gsel)
    ssel = (lax.broadcasted_iota(jnp.int32, (H, D), 1) // C
            == lax.broadcasted_iota(jnp.int32, (H, D), 0)
            ).astype(jnp.float32)
    exw = ex @ ssel
    om_ref[...] = exw * xls
    od_ref[...] = jnp.concatenate(
        [ex, jnp.zeros((ex.shape[0], H), jnp.float32)], axis=1)


def _edge_math(xlrd, efeat_pad, We_l, att_flat):
    blk = 2048
    return pl.pallas_call(
        _edge_math_body,
        out_shape=(
            jax.ShapeDtypeStruct((E_PAD, D), jnp.float32),
            jax.ShapeDtypeStruct((E_PAD, 2 * H), jnp.float32),
        ),
        grid=(E_PAD // blk,),
        in_specs=[
            pl.BlockSpec((1, blk, D), lambda i: (0, i, 0)),
            pl.BlockSpec((1, blk, D), lambda i: (1, i, 0)),
            pl.BlockSpec((blk, EH), lambda i: (i, 0)),
            pl.BlockSpec((EH, D), lambda i: (0, 0)),
            pl.BlockSpec((D,), lambda i: (0,)),
        ],
        out_specs=(
            pl.BlockSpec((blk, D), lambda i: (i, 0)),
            pl.BlockSpec((blk, 2 * H), lambda i: (i, 0)),
        ),
    )(xlrd, xlrd, efeat_pad, We_l, att_flat)


def _combine_body(a_ref, b_ref, c_ref, d_ref, h_ref, gb_ref, g_ref,
                  beta_ref, o_ref):
    msg = a_ref[0] + b_ref[0]
    den = c_ref[0][:, :H] + d_ref[0][:, :H]
    ssel = (lax.broadcasted_iota(jnp.int32, (H, D), 1) // C
            == lax.broadcasted_iota(jnp.int32, (H, D), 0)
            ).astype(jnp.float32)
    denw = den @ ssel
    y = msg / (denw + 1e-16) + gb_ref[...] + h_ref[...]
    o_ref[...] = _ln_relu(y, g_ref[...], beta_ref[...])


def _combine(acc_m, acc_d, h, gb_l, lng_l, lnb_l):
    blk = 2048
    return pl.pallas_call(
        _combine_body,
        out_shape=jax.ShapeDtypeStruct((NP, D), jnp.float32),
        grid=(NP // blk,),
        in_specs=[
            pl.BlockSpec((1, blk, D), lambda i: (0, i, 0)),
            pl.BlockSpec((1, blk, D), lambda i: (1, i, 0)),
            pl.BlockSpec((1, blk, D), lambda i: (0, i, 0)),
            pl.BlockSpec((1, blk, D), lambda i: (1, i, 0)),
            pl.BlockSpec((blk, D), lambda i: (i, 0)),
            pl.BlockSpec((D,), lambda i: (0,)),
            pl.BlockSpec((D,), lambda i: (0,)),
            pl.BlockSpec((D,), lambda i: (0,)),
        ],
        out_specs=pl.BlockSpec((blk, D), lambda i: (i, 0)),
    )(acc_m, acc_m, acc_d, acc_d, h, gb_l, lng_l, lnb_l)


def _head_body(h_ref, w1_ref, b1_ref, w2_ref, b2_ref, o_ref):
    z = jnp.maximum(h_ref[...] @ w1_ref[...] + b1_ref[...], 0.0)
    o_ref[...] = jax.nn.sigmoid(z @ w2_ref[...] + b2_ref[...])


def _head(h, h1_W, h1_b, h2_W, h2_b):
    out = pl.pallas_call(
        _head_body,
        out_shape=jax.ShapeDtypeStruct((N, 1), jnp.float32),
    )(h, h1_W, h1_b, h2_W, h2_b)
    return out[:, 0]


def _gather_body(xl_h, xr_h, src_h, dst_h, xlr_h,
                 src_all, dst_all, xlr_rows, sg0, sg1, sw0, sw1):
    c = lax.axis_index("c")
    s = lax.axis_index("s")
    w = s * NC + c
    pltpu.sync_copy(src_h.at[pl.ds(w * NB, NB)], src_all)
    pltpu.sync_copy(dst_h.at[pl.ds(w * NB, NB)], dst_all)
    sg = (sg0, sg1)
    sw = (sw0, sw1)

    def start_gather(b, slot):
        pltpu.async_copy(
            xl_h.at[src_all.at[b]], xlr_rows.at[slot, 0], sg[slot])
        pltpu.async_copy(
            xr_h.at[dst_all.at[b]], xlr_rows.at[slot, 1], sg[slot])

    def wait_wb(slot):
        pltpu.make_async_copy(
            xlr_rows.at[slot, 0], xlr_h.at[0].at[pl.ds(0, B)],
            sw[slot]).wait()
        pltpu.make_async_copy(
            xlr_rows.at[slot, 1], xlr_h.at[1].at[pl.ds(0, B)],
            sw[slot]).wait()

    start_gather(0, 0)

    def outer(j, carry):
        for k in range(2):
            b = j * 2 + k

            @pl.when(b >= 1)
            def _wait_prev_wb():
                wait_wb(1 - k)

            @pl.when(b + 1 < NB)
            def _start_next():
                start_gather(b + 1, 1 - k)

            pltpu.make_async_copy(
                xl_h.at[pl.ds(0, B)], xlr_rows.at[k, 0], sg[k]).wait()
            pltpu.make_async_copy(
                xl_h.at[pl.ds(0, B)], xlr_rows.at[k, 1], sg[k]).wait()
            pltpu.async_copy(
                xlr_rows.at[k, 0],
                xlr_h.at[0].at[pl.ds((w * NB + b) * B, B)], sw[k])
            pltpu.async_copy(
                xlr_rows.at[k, 1],
                xlr_h.at[1].at[pl.ds((w * NB + b) * B, B)], sw[k])
        return carry

    lax.fori_loop(0, NB // 2, outer, 0, unroll=False)
    wait_wb(1)


def _gather(xl_p, xr_p, src2d, dst2d):
    mesh = plsc.VectorSubcoreMesh(core_axis_name="c", subcore_axis_name="s")
    k = pl.kernel(
        _gather_body,
        out_type=jax.ShapeDtypeStruct((2, E_PAD, D), jnp.float32),
        mesh=mesh,
        scratch_types=[
            pltpu.VMEM((NB, B), jnp.int32),
            pltpu.VMEM((NB, B), jnp.int32),
            pltpu.VMEM((2, 2, B, D), jnp.float32),
            pltpu.SemaphoreType.DMA,
            pltpu.SemaphoreType.DMA,
            pltpu.SemaphoreType.DMA,
            pltpu.SemaphoreType.DMA,
        ],
    )
    return k(xl_p, xr_p, src2d, dst2d)


def _scatter_body(con_h, dst_h, z_h, out_h, dst_all, con_rows, acc,
                  sl0, sl1, ss0, ss1):
    c = lax.axis_index("c")
    s = lax.axis_index("s")
    w = s * NC + c
    pltpu.sync_copy(z_h.at[pl.ds(s * ROWS_PER_SUB, ROWS_PER_SUB)],
                    acc.at[pl.ds(s * ROWS_PER_SUB, ROWS_PER_SUB)])
    pltpu.sync_copy(dst_h.at[pl.ds(w * NB, NB)], dst_all)
    plsc.subcore_barrier()
    sl = (sl0, sl1)
    ss = (ss0, ss1)

    def start_load(b, slot):
        pltpu.async_copy(
            con_h.at[pl.ds((w * NB + b) * B, B)], con_rows.at[slot], sl[slot])

    def wait_scatter(slot):
        pltpu.make_async_copy(
            con_rows.at[slot], acc.at[pl.ds(0, B)], ss[slot]).wait()

    start_load(0, 0)

    def outer(j, carry):
        for k in range(2):
            b = j * 2 + k

            @pl.when(b >= 1)
            def _wait_prev():
                wait_scatter(1 - k)

            @pl.when(b + 1 < NB)
            def _start_next():
                start_load(b + 1, 1 - k)

            pltpu.make_async_copy(
                con_h.at[pl.ds(0, B)], con_rows.at[k], sl[k]).wait()
            pltpu.async_copy(
                con_rows.at[k], acc.at[dst_all.at[b]], ss[k], add=True)
        return carry

    lax.fori_loop(0, NB // 2, outer, 0, unroll=False)
    wait_scatter(1)
    plsc.subcore_barrier()
    pltpu.sync_copy(acc.at[pl.ds(s * ROWS_PER_SUB, ROWS_PER_SUB)],
                    out_h.at[c].at[pl.ds(s * ROWS_PER_SUB, ROWS_PER_SUB)])


def _scatter(contrib, dst2d, zeros_acc):
    mesh = plsc.VectorSubcoreMesh(core_axis_name="c", subcore_axis_name="s")
    k = pl.kernel(
        _scatter_body,
        out_type=jax.ShapeDtypeStruct((NC, NP, D), jnp.float32),
        mesh=mesh,
        scratch_types=[
            pltpu.VMEM((NB, B), jnp.int32),
            pltpu.VMEM((2, B, D), jnp.float32),
            pltpu.VMEM_SHARED((NP, D), jnp.float32),
            pltpu.SemaphoreType.DMA,
            pltpu.SemaphoreType.DMA,
            pltpu.SemaphoreType.DMA,
            pltpu.SemaphoreType.DMA,
        ],
    )
    return k(contrib, dst2d, zeros_acc)


def _scatter_den_body(pk_h, dst_h, z_h, out_h, dst_all, pk, exbuf, acc,
                      sp0, sp1, ss0, ss1):
    c = lax.axis_index("c")
    s = lax.axis_index("s")
    w = s * NC + c
    pltpu.sync_copy(z_h.at[pl.ds(s * ROWS_PER_SUB, ROWS_PER_SUB)],
                    acc.at[pl.ds(s * ROWS_PER_SUB, ROWS_PER_SUB)])
    pltpu.sync_copy(dst_h.at[pl.ds(w * NB, NB)], dst_all)
    pltpu.sync_copy(z_h.at[pl.ds(0, B)], exbuf.at[0])
    pltpu.sync_copy(z_h.at[pl.ds(0, B)], exbuf.at[1])
    plsc.subcore_barrier()
    sp = (sp0, sp1)
    ss = (ss0, ss1)

    def start_load(b, slot):
        pltpu.async_copy(
            pk_h.at[pl.ds((w * NB + b) * (B // 8), B // 8)], pk.at[slot],
            sp[slot])

    def wait_scatter(slot):
        pltpu.make_async_copy(
            exbuf.at[slot], acc.at[pl.ds(0, B)], ss[slot]).wait()

    start_load(0, 0)

    def outer(j, carry):
        for k in range(2):
            b = j * 2 + k

            @pl.when(b + 1 < NB)
            def _start_next():
                start_load(b + 1, 1 - k)

            pltpu.make_async_copy(
                pk_h.at[pl.ds(0, B // 8)], pk.at[k], sp[k]).wait()

            @pl.when(b >= 2)
            def _wait_prev():
                wait_scatter(k)

            def expand(e, carry2):
                exbuf[k, e, pl.ds(0, 16)] = pk[k, e // 8, pl.ds((e % 8) * 16, 16)]
                return carry2

            lax.fori_loop(0, B, expand, 0, unroll=8)
            pltpu.async_copy(
                exbuf.at[k], acc.at[dst_all.at[b]], ss[k], add=True)
        return carry

    lax.fori_loop(0, NB // 2, outer, 0, unroll=False)
    wait_scatter(0)
    wait_scatter(1)
    plsc.subcore_barrier()
    pltpu.sync_copy(acc.at[pl.ds(s * ROWS_PER_SUB, ROWS_PER_SUB)],
                    out_h.at[c].at[pl.ds(s * ROWS_PER_SUB, ROWS_PER_SUB)])


def _scatter_den(pk2d, dst2d, zeros_acc):
    mesh = plsc.VectorSubcoreMesh(core_axis_name="c", subcore_axis_name="s")
    k = pl.kernel(
        _scatter_den_body,
        out_type=jax.ShapeDtypeStruct((NC, NP, D), jnp.float32),
        mesh=mesh,
        scratch_types=[
            pltpu.VMEM((NB, B), jnp.int32),
            pltpu.VMEM((2, B // 8, D), jnp.float32),
            pltpu.VMEM((2, B, D), jnp.float32),
            pltpu.VMEM_SHARED((NP, D), jnp.float32),
            pltpu.SemaphoreType.DMA,
            pltpu.SemaphoreType.DMA,
            pltpu.SemaphoreType.DMA,
            pltpu.SemaphoreType.DMA,
        ],
    )
    return k(pk2d, dst2d, zeros_acc)


def kernel(x, edge_index, edge_attr, ne_W, ne_b, ne_g, ne_beta, ee_W, ee_b,
           ee_g, ee_beta, Wl, bl, Wr, br, We, att, gb, ln_g, ln_b, h1_W,
           h1_b, h2_W, h2_b):
    src = edge_index[0]
    dst = edge_index[1]
    src_f = jnp.concatenate([src, jnp.zeros((E_PAD - E,), jnp.int32)])
    dst_f = jnp.concatenate([dst, jnp.full((E_PAD - E,), N, jnp.int32)])
    src_p = src_f.reshape(E_PAD // B, B)
    dst_p = dst_f.reshape(E_PAD // B, B)
    ea_p = jnp.concatenate(
        [edge_attr, jnp.zeros((E_PAD - E, 3), jnp.float32)], axis=0)
    x_p = jnp.concatenate(
        [x, jnp.zeros((NP - N, x.shape[1]), jnp.float32)], axis=0)

    h = _node_encoder(x_p, ne_W, ne_b, ne_g, ne_beta)
    efeat = _edge_encoder(ea_p, ee_W, ee_b, ee_g, ee_beta)
    zeros_acc = jnp.zeros((NP, D), jnp.float32)

    for l in range(6):
        xl_p, xr_p = _projections(h, Wl[l], bl[l], Wr[l], br[l])
        xlrd = _gather(xl_p, xr_p, src_p, dst_p)
        con_m, con_d = _edge_math(xlrd, efeat, We[l], att[l].reshape(D))
        acc_m = _scatter(con_m, dst_p, zeros_acc)
        acc_d = _scatter_den(con_d.reshape(E_PAD // 8, D), dst_p, zeros_acc)
        h = _combine(acc_m, acc_d, h, gb[l], ln_g[l], ln_b[l])

    return _head(h[:N], h1_W, h1_b, h2_W, h2_b)


# speedup vs baseline: 1.0492x; 1.0492x over previous
import jax
import jax.numpy as jnp
from jax import lax
from jax.experimental import pallas as pl
from jax.experimental.pallas import tpu as pltpu
from jax.experimental.pallas import tpu_sc as plsc

N = 10000
E = 160000
H = 8
C = 16
D = 128
EH = 64

NC = 2
NS = 16
NW = NC * NS
B = 128
E_PAD = 163840
EW = E_PAD // NW
NB = EW // B
NP = 10240
ROWS_PER_SUB = NP // NS


def _ln_relu(y, g, b):
    mu = jnp.mean(y, axis=-1, keepdims=True)
    var = jnp.mean((y - mu) ** 2, axis=-1, keepdims=True)
    return jnp.maximum((y - mu) / jnp.sqrt(var + 1e-5) * g + b, 0.0)


def _node_enc_body(x_ref, w_ref, b_ref, g_ref, beta_ref, o_ref):
    y = x_ref[...] @ w_ref[...] + b_ref[...]
    o_ref[...] = _ln_relu(y, g_ref[...], beta_ref[...])


def _node_encoder(x_pad, ne_W, ne_b, ne_g, ne_beta):
    return pl.pallas_call(
        _node_enc_body,
        out_shape=jax.ShapeDtypeStruct((NP, D), jnp.float32),
    )(x_pad, ne_W, ne_b, ne_g, ne_beta)


def _edge_encoder(edge_attr_pad, ee_W, ee_b, ee_g, ee_beta):
    blk = 8192
    return pl.pallas_call(
        _node_enc_body,
        out_shape=jax.ShapeDtypeStruct((E_PAD, EH), jnp.float32),
        grid=(E_PAD // blk,),
        in_specs=[
            pl.BlockSpec((blk, 3), lambda i: (i, 0)),
            pl.BlockSpec((3, EH), lambda i: (0, 0)),
            pl.BlockSpec((EH,), lambda i: (0,)),
            pl.BlockSpec((EH,), lambda i: (0,)),
            pl.BlockSpec((EH,), lambda i: (0,)),
        ],
        out_specs=pl.BlockSpec((blk, EH), lambda i: (i, 0)),
    )(edge_attr_pad, ee_W, ee_b, ee_g, ee_beta)


def _proj_body(h_ref, wl_ref, bl_ref, wr_ref, br_ref, xl_ref, xr_ref):
    h = h_ref[...]
    xl_ref[...] = h @ wl_ref[...] + bl_ref[...]
    xr_ref[...] = h @ wr_ref[...] + br_ref[...]


def _projections(h, Wl_l, bl_l, Wr_l, br_l):
    return pl.pallas_call(
        _proj_body,
        out_shape=(
            jax.ShapeDtypeStruct((NP, D), jnp.float32),
            jax.ShapeDtypeStruct((NP, D), jnp.float32),
        ),
    )(h, Wl_l, bl_l, Wr_l, br_l)


def _edge_math_body(xls_ref, xrd_ref, ef_ref, we_ref, att_ref,
                    om_ref, od_ref):
    xls = xls_ref[0]
    ee = ef_ref[...] @ we_ref[...]
    m = xls + xrd_ref[0] + ee
    m = jnp.maximum(m, 0.2 * m)
    t = m * att_ref[...]
    ii = lax.broadcasted_iota(jnp.int32, (D, H), 1)
    jj = lax.broadcasted_iota(jnp.int32, (D, H), 0)
    gsel = (jj // C == ii).astype(jnp.float32)
    ex = jnp.exp(t @ gsel)
    ssel = (lax.broadcasted_iota(jnp.int32, (H, D), 1) // C
            == lax.broadcasted_iota(jnp.int32, (H, D), 0)
            ).astype(jnp.float32)
    exw = ex @ ssel
    om_ref[...] = exw * xls
    od_ref[...] = jnp.concatenate(
        [ex, jnp.zeros((ex.shape[0], D - H), jnp.float32)], axis=1)


def _edge_math(xlrd, efeat_pad, We_l, att_flat):
    blk = 2048
    return pl.pallas_call(
        _edge_math_body,
        out_shape=(
            jax.ShapeDtypeStruct((E_PAD, D), jnp.float32),
            jax.ShapeDtypeStruct((E_PAD, D), jnp.float32),
        ),
        grid=(E_PAD // blk,),
        in_specs=[
            pl.BlockSpec((1, blk, D), lambda i: (0, i, 0)),
            pl.BlockSpec((1, blk, D), lambda i: (1, i, 0)),
            pl.BlockSpec((blk, EH), lambda i: (i, 0)),
            pl.BlockSpec((EH, D), lambda i: (0, 0)),
            pl.BlockSpec((D,), lambda i: (0,)),
        ],
        out_specs=(
            pl.BlockSpec((blk, D), lambda i: (i, 0)),
            pl.BlockSpec((blk, D), lambda i: (i, 0)),
        ),
    )(xlrd, xlrd, efeat_pad, We_l, att_flat)


def _combine_body(a_ref, b_ref, c_ref, d_ref, h_ref, gb_ref, g_ref,
                  beta_ref, o_ref):
    msg = a_ref[0] + b_ref[0]
    den = c_ref[0][:, :H] + d_ref[0][:, :H]
    ssel = (lax.broadcasted_iota(jnp.int32, (H, D), 1) // C
            == lax.broadcasted_iota(jnp.int32, (H, D), 0)
            ).astype(jnp.float32)
    denw = den @ ssel
    y = msg / (denw + 1e-16) + gb_ref[...] + h_ref[...]
    o_ref[...] = _ln_relu(y, g_ref[...], beta_ref[...])


def _combine(acc_m, acc_d, h, gb_l, lng_l, lnb_l):
    blk = 2048
    return pl.pallas_call(
        _combine_body,
        out_shape=jax.ShapeDtypeStruct((NP, D), jnp.float32),
        grid=(NP // blk,),
        in_specs=[
            pl.BlockSpec((1, blk, D), lambda i: (0, i, 0)),
            pl.BlockSpec((1, blk, D), lambda i: (1, i, 0)),
            pl.BlockSpec((1, blk, D), lambda i: (0, i, 0)),
            pl.BlockSpec((1, blk, D), lambda i: (1, i, 0)),
            pl.BlockSpec((blk, D), lambda i: (i, 0)),
            pl.BlockSpec((D,), lambda i: (0,)),
            pl.BlockSpec((D,), lambda i: (0,)),
            pl.BlockSpec((D,), lambda i: (0,)),
        ],
        out_specs=pl.BlockSpec((blk, D), lambda i: (i, 0)),
    )(acc_m, acc_m, acc_d, acc_d, h, gb_l, lng_l, lnb_l)


def _head_body(h_ref, w1_ref, b1_ref, w2_ref, b2_ref, o_ref):
    z = jnp.maximum(h_ref[...] @ w1_ref[...] + b1_ref[...], 0.0)
    o_ref[...] = jax.nn.sigmoid(z @ w2_ref[...] + b2_ref[...])


def _head(h, h1_W, h1_b, h2_W, h2_b):
    out = pl.pallas_call(
        _head_body,
        out_shape=jax.ShapeDtypeStruct((N, 1), jnp.float32),
    )(h, h1_W, h1_b, h2_W, h2_b)
    return out[:, 0]


def _gather_body(xl_h, xr_h, src_h, dst_h, xlr_h,
                 src_all, dst_all, xlr_rows, sg0, sg1, sw0, sw1):
    c = lax.axis_index("c")
    s = lax.axis_index("s")
    w = s * NC + c
    pltpu.sync_copy(src_h.at[pl.ds(w * NB, NB)], src_all)
    pltpu.sync_copy(dst_h.at[pl.ds(w * NB, NB)], dst_all)
    sg = (sg0, sg1)
    sw = (sw0, sw1)

    def start_gather(b, slot):
        pltpu.async_copy(
            xl_h.at[src_all.at[b]], xlr_rows.at[slot, 0], sg[slot])
        pltpu.async_copy(
            xr_h.at[dst_all.at[b]], xlr_rows.at[slot, 1], sg[slot])

    def wait_wb(slot):
        pltpu.make_async_copy(
            xlr_rows.at[slot, 0], xlr_h.at[0].at[pl.ds(0, B)],
            sw[slot]).wait()
        pltpu.make_async_copy(
            xlr_rows.at[slot, 1], xlr_h.at[1].at[pl.ds(0, B)],
            sw[slot]).wait()

    start_gather(0, 0)

    def outer(j, carry):
        for k in range(2):
            b = j * 2 + k

            @pl.when(b >= 1)
            def _wait_prev_wb():
                wait_wb(1 - k)

            @pl.when(b + 1 < NB)
            def _start_next():
                start_gather(b + 1, 1 - k)

            pltpu.make_async_copy(
                xl_h.at[pl.ds(0, B)], xlr_rows.at[k, 0], sg[k]).wait()
            pltpu.make_async_copy(
                xl_h.at[pl.ds(0, B)], xlr_rows.at[k, 1], sg[k]).wait()
            pltpu.async_copy(
                xlr_rows.at[k, 0],
                xlr_h.at[0].at[pl.ds((w * NB + b) * B, B)], sw[k])
            pltpu.async_copy(
                xlr_rows.at[k, 1],
                xlr_h.at[1].at[pl.ds((w * NB + b) * B, B)], sw[k])
        return carry

    lax.fori_loop(0, NB // 2, outer, 0, unroll=False)
    wait_wb(1)


def _gather(xl_p, xr_p, src2d, dst2d):
    mesh = plsc.VectorSubcoreMesh(core_axis_name="c", subcore_axis_name="s")
    k = pl.kernel(
        _gather_body,
        out_type=jax.ShapeDtypeStruct((2, E_PAD, D), jnp.float32),
        mesh=mesh,
        scratch_types=[
            pltpu.VMEM((NB, B), jnp.int32),
            pltpu.VMEM((NB, B), jnp.int32),
            pltpu.VMEM((2, 2, B, D), jnp.float32),
            pltpu.SemaphoreType.DMA,
            pltpu.SemaphoreType.DMA,
            pltpu.SemaphoreType.DMA,
            pltpu.SemaphoreType.DMA,
        ],
    )
    return k(xl_p, xr_p, src2d, dst2d)


def _scatter_body(con_h, dst_h, z_h, out_h, dst_all, con_rows, acc,
                  sl0, sl1, ss0, ss1):
    c = lax.axis_index("c")
    s = lax.axis_index("s")
    w = s * NC + c
    pltpu.sync_copy(z_h.at[pl.ds(s * ROWS_PER_SUB, ROWS_PER_SUB)],
                    acc.at[pl.ds(s * ROWS_PER_SUB, ROWS_PER_SUB)])
    pltpu.sync_copy(dst_h.at[pl.ds(w * NB, NB)], dst_all)
    plsc.subcore_barrier()
    sl = (sl0, sl1)
    ss = (ss0, ss1)

    def start_load(b, slot):
        pltpu.async_copy(
            con_h.at[pl.ds((w * NB + b) * B, B)], con_rows.at[slot], sl[slot])

    def wait_scatter(slot):
        pltpu.make_async_copy(
            con_rows.at[slot], acc.at[pl.ds(0, B)], ss[slot]).wait()

    start_load(0, 0)

    def outer(j, carry):
        for k in range(2):
            b = j * 2 + k

            @pl.when(b >= 1)
            def _wait_prev():
                wait_scatter(1 - k)

            @pl.when(b + 1 < NB)
            def _start_next():
                start_load(b + 1, 1 - k)

            pltpu.make_async_copy(
                con_h.at[pl.ds(0, B)], con_rows.at[k], sl[k]).wait()
            pltpu.async_copy(
                con_rows.at[k], acc.at[dst_all.at[b]], ss[k], add=True)
        return carry

    lax.fori_loop(0, NB // 2, outer, 0, unroll=False)
    wait_scatter(1)
    plsc.subcore_barrier()
    pltpu.sync_copy(acc.at[pl.ds(s * ROWS_PER_SUB, ROWS_PER_SUB)],
                    out_h.at[c].at[pl.ds(s * ROWS_PER_SUB, ROWS_PER_SUB)])


def _scatter(contrib, dst2d, zeros_acc):
    mesh = plsc.VectorSubcoreMesh(core_axis_name="c", subcore_axis_name="s")
    k = pl.kernel(
        _scatter_body,
        out_type=jax.ShapeDtypeStruct((NC, NP, D), jnp.float32),
        mesh=mesh,
        scratch_types=[
            pltpu.VMEM((NB, B), jnp.int32),
            pltpu.VMEM((2, B, D), jnp.float32),
            pltpu.VMEM_SHARED((NP, D), jnp.float32),
            pltpu.SemaphoreType.DMA,
            pltpu.SemaphoreType.DMA,
            pltpu.SemaphoreType.DMA,
            pltpu.SemaphoreType.DMA,
        ],
    )
    return k(contrib, dst2d, zeros_acc)


def kernel(x, edge_index, edge_attr, ne_W, ne_b, ne_g, ne_beta, ee_W, ee_b,
           ee_g, ee_beta, Wl, bl, Wr, br, We, att, gb, ln_g, ln_b, h1_W,
           h1_b, h2_W, h2_b):
    src = edge_index[0]
    dst = edge_index[1]
    src_f = jnp.concatenate([src, jnp.zeros((E_PAD - E,), jnp.int32)])
    dst_f = jnp.concatenate([dst, jnp.full((E_PAD - E,), N, jnp.int32)])
    src_p = src_f.reshape(E_PAD // B, B)
    dst_p = dst_f.reshape(E_PAD // B, B)
    ea_p = jnp.concatenate(
        [edge_attr, jnp.zeros((E_PAD - E, 3), jnp.float32)], axis=0)
    x_p = jnp.concatenate(
        [x, jnp.zeros((NP - N, x.shape[1]), jnp.float32)], axis=0)

    h = _node_encoder(x_p, ne_W, ne_b, ne_g, ne_beta)
    efeat = _edge_encoder(ea_p, ee_W, ee_b, ee_g, ee_beta)
    zeros_acc = jnp.zeros((NP, D), jnp.float32)

    for l in range(6):
        xl_p, xr_p = _projections(h, Wl[l], bl[l], Wr[l], br[l])
        xlrd = _gather(xl_p, xr_p, src_p, dst_p)
        con_m, con_d = _edge_math(xlrd, efeat, We[l], att[l].reshape(D))
        acc_m = _scatter(con_m, dst_p, zeros_acc)
        acc_d = _scatter(con_d, dst_p, zeros_acc)
        h = _combine(acc_m, acc_d, h, gb[l], ln_g[l], ln_b[l])

    return _head(h[:N], h1_W, h1_b, h2_W, h2_b)


# speedup vs baseline: 1.0777x; 1.0272x over previous
import jax
import jax.numpy as jnp
from jax import lax
from jax.experimental import pallas as pl
from jax.experimental.pallas import tpu as pltpu
from jax.experimental.pallas import tpu_sc as plsc

N = 10000
E = 160000
H = 8
C = 16
D = 128
EH = 64

NC = 2
NS = 16
NW = NC * NS
B = 128
E_PAD = 163840
EW = E_PAD // NW
NB = EW // B
NP = 10240
ROWS_PER_SUB = NP // NS


def _ln_relu(y, g, b):
    mu = jnp.mean(y, axis=-1, keepdims=True)
    var = jnp.mean((y - mu) ** 2, axis=-1, keepdims=True)
    return jnp.maximum((y - mu) / jnp.sqrt(var + 1e-5) * g + b, 0.0)


def _node_enc_body(x_ref, w_ref, b_ref, g_ref, beta_ref, o_ref):
    y = x_ref[...] @ w_ref[...] + b_ref[...]
    o_ref[...] = _ln_relu(y, g_ref[...], beta_ref[...])


def _node_encoder(x_pad, ne_W, ne_b, ne_g, ne_beta):
    return pl.pallas_call(
        _node_enc_body,
        out_shape=jax.ShapeDtypeStruct((NP, D), jnp.float32),
    )(x_pad, ne_W, ne_b, ne_g, ne_beta)


def _edge_encoder(edge_attr_pad, ee_W, ee_b, ee_g, ee_beta):
    blk = 8192
    return pl.pallas_call(
        _node_enc_body,
        out_shape=jax.ShapeDtypeStruct((E_PAD, EH), jnp.float32),
        grid=(E_PAD // blk,),
        in_specs=[
            pl.BlockSpec((blk, 3), lambda i: (i, 0)),
            pl.BlockSpec((3, EH), lambda i: (0, 0)),
            pl.BlockSpec((EH,), lambda i: (0,)),
            pl.BlockSpec((EH,), lambda i: (0,)),
            pl.BlockSpec((EH,), lambda i: (0,)),
        ],
        out_specs=pl.BlockSpec((blk, EH), lambda i: (i, 0)),
    )(edge_attr_pad, ee_W, ee_b, ee_g, ee_beta)


def _proj_body(h_ref, wl_ref, bl_ref, wr_ref, br_ref, xl_ref, xr_ref):
    h = h_ref[...]
    xl_ref[...] = h @ wl_ref[...] + bl_ref[...]
    xr_ref[...] = h @ wr_ref[...] + br_ref[...]


def _projections(h, Wl_l, bl_l, Wr_l, br_l):
    return pl.pallas_call(
        _proj_body,
        out_shape=(
            jax.ShapeDtypeStruct((NP, D), jnp.float32),
            jax.ShapeDtypeStruct((NP, D), jnp.float32),
        ),
    )(h, Wl_l, bl_l, Wr_l, br_l)


def _edge_math_body(xls_ref, xrd_ref, ef_ref, we_ref, att_ref,
                    om_ref, od_ref):
    xls = xls_ref[...]
    ee = ef_ref[...] @ we_ref[...]
    m = xls + xrd_ref[...] + ee
    m = jnp.maximum(m, 0.2 * m)
    t = m * att_ref[...]
    ii = lax.broadcasted_iota(jnp.int32, (D, H), 1)
    jj = lax.broadcasted_iota(jnp.int32, (D, H), 0)
    gsel = (jj // C == ii).astype(jnp.float32)
    ex = jnp.exp(t @ gsel)
    ssel = (lax.broadcasted_iota(jnp.int32, (H, D), 1) // C
            == lax.broadcasted_iota(jnp.int32, (H, D), 0)
            ).astype(jnp.float32)
    exw = ex @ ssel
    om_ref[...] = exw * xls
    od_ref[...] = jnp.concatenate(
        [ex, jnp.zeros((ex.shape[0], D - H), jnp.float32)], axis=1)


def _edge_math(xl_src, xr_dst, efeat_pad, We_l, att_flat):
    blk = 2048
    return pl.pallas_call(
        _edge_math_body,
        out_shape=(
            jax.ShapeDtypeStruct((E_PAD, D), jnp.float32),
            jax.ShapeDtypeStruct((E_PAD, D), jnp.float32),
        ),
        grid=(E_PAD // blk,),
        in_specs=[
            pl.BlockSpec((blk, D), lambda i: (i, 0)),
            pl.BlockSpec((blk, D), lambda i: (i, 0)),
            pl.BlockSpec((blk, EH), lambda i: (i, 0)),
            pl.BlockSpec((EH, D), lambda i: (0, 0)),
            pl.BlockSpec((D,), lambda i: (0,)),
        ],
        out_specs=(
            pl.BlockSpec((blk, D), lambda i: (i, 0)),
            pl.BlockSpec((blk, D), lambda i: (i, 0)),
        ),
    )(xl_src, xr_dst, efeat_pad, We_l, att_flat)


def _combine_body(a_ref, b_ref, c_ref, d_ref, h_ref, gb_ref, g_ref,
                  beta_ref, o_ref):
    msg = a_ref[0] + b_ref[0]
    den = c_ref[0][:, :H] + d_ref[0][:, :H]
    ssel = (lax.broadcasted_iota(jnp.int32, (H, D), 1) // C
            == lax.broadcasted_iota(jnp.int32, (H, D), 0)
            ).astype(jnp.float32)
    denw = den @ ssel
    y = msg / (denw + 1e-16) + gb_ref[...] + h_ref[...]
    o_ref[...] = _ln_relu(y, g_ref[...], beta_ref[...])


def _combine(acc_m, acc_d, h, gb_l, lng_l, lnb_l):
    blk = 2048
    return pl.pallas_call(
        _combine_body,
        out_shape=jax.ShapeDtypeStruct((NP, D), jnp.float32),
        grid=(NP // blk,),
        in_specs=[
            pl.BlockSpec((1, blk, D), lambda i: (0, i, 0)),
            pl.BlockSpec((1, blk, D), lambda i: (1, i, 0)),
            pl.BlockSpec((1, blk, D), lambda i: (0, i, 0)),
            pl.BlockSpec((1, blk, D), lambda i: (1, i, 0)),
            pl.BlockSpec((blk, D), lambda i: (i, 0)),
            pl.BlockSpec((D,), lambda i: (0,)),
            pl.BlockSpec((D,), lambda i: (0,)),
            pl.BlockSpec((D,), lambda i: (0,)),
        ],
        out_specs=pl.BlockSpec((blk, D), lambda i: (i, 0)),
    )(acc_m, acc_m, acc_d, acc_d, h, gb_l, lng_l, lnb_l)


def _head_body(h_ref, w1_ref, b1_ref, w2_ref, b2_ref, o_ref):
    z = jnp.maximum(h_ref[...] @ w1_ref[...] + b1_ref[...], 0.0)
    o_ref[...] = jax.nn.sigmoid(z @ w2_ref[...] + b2_ref[...])


def _head(h, h1_W, h1_b, h2_W, h2_b):
    out = pl.pallas_call(
        _head_body,
        out_shape=jax.ShapeDtypeStruct((N, 1), jnp.float32),
    )(h, h1_W, h1_b, h2_W, h2_b)
    return out[:, 0]


def _gather_body(xl_h, xr_h, src_h, dst_h, xls_h, xrd_h,
                 src_all, dst_all, xl_rows, xr_rows, sg0, sg1, sw0, sw1):
    c = lax.axis_index("c")
    s = lax.axis_index("s")
    w = s * NC + c
    pltpu.sync_copy(src_h.at[pl.ds(w * NB, NB)], src_all)
    pltpu.sync_copy(dst_h.at[pl.ds(w * NB, NB)], dst_all)
    sg = (sg0, sg1)
    sw = (sw0, sw1)

    def start_gather(b, slot):
        pltpu.async_copy(xl_h.at[src_all.at[b]], xl_rows.at[slot], sg[slot])
        pltpu.async_copy(xr_h.at[dst_all.at[b]], xr_rows.at[slot], sg[slot])

    def wait_wb(slot):
        pltpu.make_async_copy(
            xl_rows.at[slot], xls_h.at[pl.ds(0, B)], sw[slot]).wait()
        pltpu.make_async_copy(
            xr_rows.at[slot], xrd_h.at[pl.ds(0, B)], sw[slot]).wait()

    start_gather(0, 0)

    def outer(j, carry):
        for k in range(2):
            b = j * 2 + k

            @pl.when(b >= 1)
            def _wait_prev_wb():
                wait_wb(1 - k)

            @pl.when(b + 1 < NB)
            def _start_next():
                start_gather(b + 1, 1 - k)

            pltpu.make_async_copy(
                xl_h.at[pl.ds(0, B)], xl_rows.at[k], sg[k]).wait()
            pltpu.make_async_copy(
                xl_h.at[pl.ds(0, B)], xr_rows.at[k], sg[k]).wait()
            pltpu.async_copy(
                xl_rows.at[k], xls_h.at[pl.ds((w * NB + b) * B, B)], sw[k])
            pltpu.async_copy(
                xr_rows.at[k], xrd_h.at[pl.ds((w * NB + b) * B, B)], sw[k])
        return carry

    lax.fori_loop(0, NB // 2, outer, 0, unroll=False)
    wait_wb(1)


def _gather(xl_p, xr_p, src2d, dst2d):
    mesh = plsc.VectorSubcoreMesh(core_axis_name="c", subcore_axis_name="s")
    k = pl.kernel(
        _gather_body,
        out_type=(
            jax.ShapeDtypeStruct((E_PAD, D), jnp.float32),
            jax.ShapeDtypeStruct((E_PAD, D), jnp.float32),
        ),
        mesh=mesh,
        scratch_types=[
            pltpu.VMEM((NB, B), jnp.int32),
            pltpu.VMEM((NB, B), jnp.int32),
            pltpu.VMEM((2, B, D), jnp.float32),
            pltpu.VMEM((2, B, D), jnp.float32),
            pltpu.SemaphoreType.DMA,
            pltpu.SemaphoreType.DMA,
            pltpu.SemaphoreType.DMA,
            pltpu.SemaphoreType.DMA,
        ],
    )
    return k(xl_p, xr_p, src2d, dst2d)


def _scatter_body(con_h, dst_h, z_h, out_h, dst_all, con_rows, acc,
                  sl0, sl1, ss0, ss1):
    c = lax.axis_index("c")
    s = lax.axis_index("s")
    w = s * NC + c
    pltpu.sync_copy(z_h.at[pl.ds(s * ROWS_PER_SUB, ROWS_PER_SUB)],
                    acc.at[pl.ds(s * ROWS_PER_SUB, ROWS_PER_SUB)])
    pltpu.sync_copy(dst_h.at[pl.ds(w * NB, NB)], dst_all)
    plsc.subcore_barrier()
    sl = (sl0, sl1)
    ss = (ss0, ss1)

    def start_load(b, slot):
        pltpu.async_copy(
            con_h.at[pl.ds((w * NB + b) * B, B)], con_rows.at[slot], sl[slot])

    def wait_scatter(slot):
        pltpu.make_async_copy(
            con_rows.at[slot], acc.at[pl.ds(0, B)], ss[slot]).wait()

    start_load(0, 0)

    def outer(j, carry):
        for k in range(2):
            b = j * 2 + k

            @pl.when(b >= 1)
            def _wait_prev():
                wait_scatter(1 - k)

            @pl.when(b + 1 < NB)
            def _start_next():
                start_load(b + 1, 1 - k)

            pltpu.make_async_copy(
                con_h.at[pl.ds(0, B)], con_rows.at[k], sl[k]).wait()
            pltpu.async_copy(
                con_rows.at[k], acc.at[dst_all.at[b]], ss[k], add=True)
        return carry

    lax.fori_loop(0, NB // 2, outer, 0, unroll=False)
    wait_scatter(1)
    plsc.subcore_barrier()
    pltpu.sync_copy(acc.at[pl.ds(s * ROWS_PER_SUB, ROWS_PER_SUB)],
                    out_h.at[c].at[pl.ds(s * ROWS_PER_SUB, ROWS_PER_SUB)])


def _scatter(contrib, dst2d, zeros_acc):
    mesh = plsc.VectorSubcoreMesh(core_axis_name="c", subcore_axis_name="s")
    k = pl.kernel(
        _scatter_body,
        out_type=jax.ShapeDtypeStruct((NC, NP, D), jnp.float32),
        mesh=mesh,
        scratch_types=[
            pltpu.VMEM((NB, B), jnp.int32),
            pltpu.VMEM((2, B, D), jnp.float32),
            pltpu.VMEM_SHARED((NP, D), jnp.float32),
            pltpu.SemaphoreType.DMA,
            pltpu.SemaphoreType.DMA,
            pltpu.SemaphoreType.DMA,
            pltpu.SemaphoreType.DMA,
        ],
    )
    return k(contrib, dst2d, zeros_acc)


def kernel(x, edge_index, edge_attr, ne_W, ne_b, ne_g, ne_beta, ee_W, ee_b,
           ee_g, ee_beta, Wl, bl, Wr, br, We, att, gb, ln_g, ln_b, h1_W,
           h1_b, h2_W, h2_b):
    src = edge_index[0]
    dst = edge_index[1]
    src_f = jnp.concatenate([src, jnp.zeros((E_PAD - E,), jnp.int32)])
    dst_f = jnp.concatenate([dst, jnp.full((E_PAD - E,), N, jnp.int32)])
    src_p = src_f.reshape(E_PAD // B, B)
    dst_p = dst_f.reshape(E_PAD // B, B)
    ea_p = jnp.concatenate(
        [edge_attr, jnp.zeros((E_PAD - E, 3), jnp.float32)], axis=0)
    x_p = jnp.concatenate(
        [x, jnp.zeros((NP - N, x.shape[1]), jnp.float32)], axis=0)

    h = _node_encoder(x_p, ne_W, ne_b, ne_g, ne_beta)
    efeat = _edge_encoder(ea_p, ee_W, ee_b, ee_g, ee_beta)
    zeros_acc = jnp.zeros((NP, D), jnp.float32)

    for l in range(6):
        xl_p, xr_p = _projections(h, Wl[l], bl[l], Wr[l], br[l])
        xl_src, xr_dst = _gather(xl_p, xr_p, src_p, dst_p)
        con_m, con_d = _edge_math(xl_src, xr_dst, efeat, We[l],
                                  att[l].reshape(D))
        acc_m = _scatter(con_m, dst_p, zeros_acc)
        acc_d = _scatter(con_d, dst_p, zeros_acc)
        h = _combine(acc_m, acc_d, h, gb[l], ln_g[l], ln_b[l])

    return _head(h[:N], h1_W, h1_b, h2_W, h2_b)
